# baseline (speedup 1.0000x reference)
import functools

import jax
import jax.numpy as jnp
from jax import lax
from jax.experimental import pallas as pl
from jax.experimental.pallas import tpu as pltpu
from jax.experimental.pallas import tpu_sc as plsc

_GB = 2
_CHUNK = 40


@functools.cache
def _gather_fn(V, D, B, H):
  info = plsc.get_sparse_core_info()
  NC, NS = info.num_cores, info.num_subcores
  NW = NC * NS
  BPW = B // NW
  assert B % NW == 0 and BPW % _GB == 0 and H % _CHUNK == 0
  G = BPW // _GB
  assert G >= 4 and G % 2 == 0
  K = _GB * H // _CHUNK
  mesh = plsc.VectorSubcoreMesh(core_axis_name="c", subcore_axis_name="s")

  @functools.partial(
      pl.kernel, mesh=mesh,
      out_type=jax.ShapeDtypeStruct((B, H, D), jnp.float32),
      compiler_params=pltpu.CompilerParams(use_tc_tiling_on_sc=False),
      scratch_types=[
          pltpu.VMEM((2, _GB, H), jnp.int32),
          pltpu.VMEM((2, _GB, H, D), jnp.float32),
          pltpu.SemaphoreType.DMA,
          pltpu.SemaphoreType.DMA,
          pltpu.SemaphoreType.DMA,
          pltpu.SemaphoreType.DMA,
          pltpu.SemaphoreType.DMA,
          pltpu.SemaphoreType.DMA,
      ],
  )
  def k(x_hbm, table_hbm, out_hbm, idx_v, rows_v,
        isem0, isem1, gsem0, gsem1, wsem0, wsem1):
    isems, gsems, wsems = (isem0, isem1), (gsem0, gsem1), (wsem0, wsem1)
    wid = lax.axis_index("s") * NC + lax.axis_index("c")
    b0 = wid * BPW

    def idx_start(h, b):
      r = jnp.where(h < G, b0 + h * _GB, b0)
      pltpu.async_copy(x_hbm.at[pl.ds(r, _GB)], idx_v.at[b], isems[b])

    def idx_wait(b):
      pltpu.make_async_copy(
          x_hbm.at[pl.ds(b0, _GB)], idx_v.at[b], isems[b]).wait()

    def chunk_refs(b, j):
      r, c = (j * _CHUNK) // H, (j * _CHUNK) % H
      return (idx_v.at[b, r, pl.ds(c, _CHUNK)],
              rows_v.at[b, r, pl.ds(c, _CHUNK)])

    def fire(b):
      for j in range(K):
        iv, rv = chunk_refs(b, j)
        pltpu.async_copy(table_hbm.at[iv], rv, gsems[b])

    def drain(b):
      for j in range(K):
        iv, rv = chunk_refs(b, j)
        pltpu.make_async_copy(table_hbm.at[iv], rv, gsems[b]).wait()

    def wb_start(g, b):
      pltpu.async_copy(
          rows_v.at[b], out_hbm.at[pl.ds(b0 + g * _GB, _GB)], wsems[b])

    def wb_wait(b):
      pltpu.make_async_copy(
          rows_v.at[b], out_hbm.at[pl.ds(b0, _GB)], wsems[b]).wait()

    idx_start(0, 0)
    idx_wait(0)
    fire(0)
    idx_start(1, 1)
    drain(0)
    wb_start(0, 0)
    idx_wait(1)
    fire(1)
    idx_start(2, 0)

    @pl.loop(0, (G - 2) // 2)
    def _(i):
      for g_off, b in ((0, 1), (1, 0)):
        g = 1 + 2 * i + g_off
        ob = 1 - b
        drain(b)
        wb_start(g, b)
        idx_wait(ob)
        wb_wait(ob)
        fire(ob)
        idx_start(g + 2, b)

    drain(1)
    wb_start(G - 1, 1)
    idx_wait(0)
    wb_wait(0)
    wb_wait(1)

  return k


def kernel(x, table):
  B, H = x.shape
  V, D = table.shape
  return _gather_fn(V, D, B, H)(x, table)


# speedup vs baseline: 1.0012x; 1.0012x over previous
import functools

import jax
import jax.numpy as jnp
from jax import lax
from jax.experimental import pallas as pl
from jax.experimental.pallas import tpu as pltpu
from jax.experimental.pallas import tpu_sc as plsc

_GB = 2
_SPLITS = ((0, 128), (128, 72))


@functools.cache
def _gather_fn(V, D, B, H):
  info = plsc.get_sparse_core_info()
  NC, NS = info.num_cores, info.num_subcores
  NW = NC * NS
  BPW = B // NW
  assert B % NW == 0 and BPW % _GB == 0
  G = BPW // _GB
  assert G >= 4 and G % 2 == 0
  assert sum(n for _, n in _SPLITS) == H
  mesh = plsc.VectorSubcoreMesh(core_axis_name="c", subcore_axis_name="s")

  @functools.partial(
      pl.kernel, mesh=mesh,
      out_type=jax.ShapeDtypeStruct((B, H, D), jnp.float32),
      compiler_params=pltpu.CompilerParams(use_tc_tiling_on_sc=False),
      scratch_types=[
          pltpu.VMEM((2, _GB, H), jnp.int32),
          pltpu.VMEM((2, _GB, H, D), jnp.float32),
          pltpu.SemaphoreType.DMA,
          pltpu.SemaphoreType.DMA,
          pltpu.SemaphoreType.DMA,
          pltpu.SemaphoreType.DMA,
          pltpu.SemaphoreType.DMA,
          pltpu.SemaphoreType.DMA,
      ],
  )
  def k(x_hbm, table_hbm, out_hbm, idx_v, rows_v,
        isem0, isem1, gsem0, gsem1, wsem0, wsem1):
    isems, gsems, wsems = (isem0, isem1), (gsem0, gsem1), (wsem0, wsem1)
    wid = lax.axis_index("s") * NC + lax.axis_index("c")
    b0 = wid * BPW

    def idx_start(h, b):
      r = jnp.where(h < G, b0 + h * _GB, b0)
      pltpu.async_copy(x_hbm.at[pl.ds(r, _GB)], idx_v.at[b], isems[b])

    def idx_wait(b):
      pltpu.make_async_copy(
          x_hbm.at[pl.ds(b0, _GB)], idx_v.at[b], isems[b]).wait()

    def chunks(b):
      for r in range(_GB):
        for c, n in _SPLITS:
          yield (idx_v.at[b, r, pl.ds(c, n)], rows_v.at[b, r, pl.ds(c, n)])

    def fire(b):
      for iv, rv in chunks(b):
        pltpu.async_copy(table_hbm.at[iv], rv, gsems[b])

    def drain(b):
      for iv, rv in chunks(b):
        pltpu.make_async_copy(table_hbm.at[iv], rv, gsems[b]).wait()

    def wb_start(g, b):
      pltpu.async_copy(
          rows_v.at[b], out_hbm.at[pl.ds(b0 + g * _GB, _GB)], wsems[b])

    def wb_wait(b):
      pltpu.make_async_copy(
          rows_v.at[b], out_hbm.at[pl.ds(b0, _GB)], wsems[b]).wait()

    idx_start(0, 0)
    idx_wait(0)
    fire(0)
    idx_start(1, 1)
    drain(0)
    wb_start(0, 0)
    idx_wait(1)
    fire(1)
    idx_start(2, 0)

    @pl.loop(0, (G - 2) // 2)
    def _(i):
      for g_off, b in ((0, 1), (1, 0)):
        g = 1 + 2 * i + g_off
        ob = 1 - b
        drain(b)
        wb_start(g, b)
        idx_wait(ob)
        wb_wait(ob)
        fire(ob)
        idx_start(g + 2, b)

    drain(1)
    wb_start(G - 1, 1)
    idx_wait(0)
    wb_wait(0)
    wb_wait(1)

  return k


def kernel(x, table):
  B, H = x.shape
  V, D = table.shape
  return _gather_fn(V, D, B, H)(x, table)
